# Initial kernel scaffold; baseline (speedup 1.0000x reference)
import jax
import jax.numpy as jnp
from jax.experimental import pallas as pl


def kernel(x, pitch_target, energy_target, params):
    raise NotImplementedError("write your pallas kernel here")


# speedup vs baseline: 26.2755x; 26.2755x over previous
import jax
import jax.numpy as jnp
from jax.experimental import pallas as pl


def _shift_down(y):
    return jnp.concatenate([jnp.zeros((1, y.shape[1]), y.dtype), y[:-1]], axis=0)


def _shift_up(y):
    return jnp.concatenate([y[1:], jnp.zeros((1, y.shape[1]), y.dtype)], axis=0)


def _conv3(h, w_ref):
    y0 = jnp.dot(h, w_ref[0], preferred_element_type=jnp.float32)
    y1 = jnp.dot(h, w_ref[1], preferred_element_type=jnp.float32)
    y2 = jnp.dot(h, w_ref[2], preferred_element_type=jnp.float32)
    return _shift_down(y0) + y1 + _shift_up(y2)


def _layer_norm(h, g, b):
    m = jnp.mean(h, axis=-1, keepdims=True)
    v = jnp.mean((h - m) ** 2, axis=-1, keepdims=True)
    return (h - m) * jax.lax.rsqrt(v + 1e-5) * g + b


def _predictor(xb, w1, b1, g1, be1, w2, b2, g2, be2, wl, bl):
    h = _conv3(xb, w1) + b1[...]
    h = jnp.maximum(h, 0.0)
    h = _layer_norm(h, g1[...], be1[...])
    h = _conv3(h, w2) + b2[...]
    h = jnp.maximum(h, 0.0)
    h = _layer_norm(h, g2[...], be2[...])
    return jnp.dot(h, wl[...], preferred_element_type=jnp.float32) + bl[0, 0]


def _body(x_ref, pt_ref, et_ref,
          p_w1, p_b1, p_g1, p_be1, p_w2, p_b2, p_g2, p_be2, p_wl, p_bl,
          e_w1, e_b1, e_g1, e_be1, e_w2, e_b2, e_g2, e_be2, e_wl, e_bl,
          pbins_ref, ebins_ref, pemb_ref, eemb_ref,
          xout_ref, ppred_ref, epred_ref):
    xb = x_ref[0]
    T, D = xb.shape

    ppred_ref[0] = _predictor(xb, p_w1, p_b1, p_g1, p_be1,
                              p_w2, p_b2, p_g2, p_be2, p_wl, p_bl)
    epred_ref[0] = _predictor(xb, e_w1, e_b1, e_g1, e_be1,
                              e_w2, e_b2, e_g2, e_be2, e_wl, e_bl)

    lane = jax.lax.broadcasted_iota(jnp.int32, (T, D), 1)

    def embed_add(t_ref, bins_ref, emb_ref):
        tcol = t_ref[0, 0].reshape(T, 1)
        cnt = jnp.sum((bins_ref[...] < tcol).astype(jnp.int32), axis=1,
                      keepdims=True)
        onehot = (lane == cnt).astype(jnp.float32)
        return jnp.dot(onehot, emb_ref[...], preferred_element_type=jnp.float32)

    xout_ref[0] = (xb + embed_add(pt_ref, pbins_ref, pemb_ref)
                   + embed_add(et_ref, ebins_ref, eemb_ref))


def kernel(x, pitch_target, energy_target, params):
    B, T, D = x.shape
    pp, ep = params["pitch_pred"], params["energy_pred"]

    def vec(v):
        return v.reshape(1, -1)

    pbins = jnp.full((1, D), 2.0, jnp.float32).at[0, : params["pitch_bins"].shape[0]].set(
        params["pitch_bins"])
    ebins = jnp.full((1, D), 2.0, jnp.float32).at[0, : params["energy_bins"].shape[0]].set(
        params["energy_bins"])

    grid = (B,)
    seq_spec = pl.BlockSpec((1, T, D), lambda b: (b, 0, 0))
    tgt_spec = pl.BlockSpec((1, 1, T), lambda b: (b, 0, 0))

    def full(a):
        return pl.BlockSpec(a.shape, lambda b: (0,) * a.ndim)

    consts = [pp["W1"], vec(pp["b1"]), vec(pp["g1"]), vec(pp["be1"]),
              pp["W2"], vec(pp["b2"]), vec(pp["g2"]), vec(pp["be2"]),
              pp["Wl"], pp["bl"].reshape(1, 1),
              ep["W1"], vec(ep["b1"]), vec(ep["g1"]), vec(ep["be1"]),
              ep["W2"], vec(ep["b2"]), vec(ep["g2"]), vec(ep["be2"]),
              ep["Wl"], ep["bl"].reshape(1, 1),
              pbins, ebins, params["pitch_embed"], params["energy_embed"]]

    out = pl.pallas_call(
        _body,
        grid=grid,
        in_specs=[seq_spec, tgt_spec, tgt_spec] + [full(c) for c in consts],
        out_specs=[seq_spec,
                   pl.BlockSpec((1, T, 1), lambda b: (b, 0, 0)),
                   pl.BlockSpec((1, T, 1), lambda b: (b, 0, 0))],
        out_shape=[jax.ShapeDtypeStruct((B, T, D), jnp.float32),
                   jax.ShapeDtypeStruct((B, T, 1), jnp.float32),
                   jax.ShapeDtypeStruct((B, T, 1), jnp.float32)],
    )(x, pitch_target.reshape(B, 1, T), energy_target.reshape(B, 1, T), *consts)

    x_out, ppred, epred = out
    return (x_out, ppred.reshape(B, T), epred.reshape(B, T))
